# baseline (speedup 1.0000x reference)
import functools

import jax
import jax.numpy as jnp
import numpy as np
from jax.experimental import pallas as pl

NUM_CLASSES = 80
REG_MAX = 16
STRIDES = (8, 16, 32)
SHAPES = ((80, 80), (40, 40), (20, 20))

ANCHOR_TILE = 400


def _host_constants():
    anchor_rows = []
    stride_rows = []
    for (h, w), s in zip(SHAPES, STRIDES):
        xs = np.arange(w, dtype=np.float32) + 0.5
        ys = np.arange(h, dtype=np.float32) + 0.5
        gx = np.broadcast_to(xs[None, :], (h, w)).reshape(-1)
        gy = np.broadcast_to(ys[:, None], (h, w)).reshape(-1)
        anchor_rows.append(np.stack([gx, gy], axis=1))
        stride_rows.append(np.full((h * w, 1), float(s), dtype=np.float32))
    anchors = np.concatenate(anchor_rows, axis=0)
    strides = np.concatenate(stride_rows, axis=0)

    wmat = np.zeros((64, 8), dtype=np.float32)
    for c in range(64):
        side, r = divmod(c, REG_MAX)
        wmat[c, side] = 1.0
        wmat[c, 4 + side] = float(r)
    return anchors, strides, wmat


_ANCHORS, _STRIDES, _WMAT = _host_constants()


def _decode_block(x, anc, strd, wmat, out_ref):
    bb, t, _ = x.shape
    box = x[:, :, :64]
    e = jnp.exp(box)
    r = jax.lax.dot_general(
        e.reshape(bb * t, 64), wmat,
        (((1,), (0,)), ((), ())),
        preferred_element_type=jnp.float32,
    ).reshape(bb, t, 8)
    dist = r[:, :, 4:] / r[:, :, :4]
    lt = dist[:, :, :2]
    rb = dist[:, :, 2:]
    cxy = anc[None] + (rb - lt) * 0.5
    wh = lt + rb
    dbox = jnp.concatenate([cxy, wh], axis=2) * strd[None]
    out_ref[:, :, :4] = dbox
    out_ref[:, :, 4:] = jax.nn.sigmoid(x[:, :, 64:])


def _body(n_tiles, s8_ref, s16_ref, s32_ref, anc_ref, strd_ref, w_ref,
          out_ref):
    t = pl.program_id(1)
    anc = anc_ref[...]
    strd = strd_ref[...]
    wmat = w_ref[...]

    @pl.when(t < n_tiles[0])
    def _():
        _decode_block(s8_ref[...], anc, strd, wmat, out_ref)

    @pl.when(jnp.logical_and(t >= n_tiles[0], t < n_tiles[0] + n_tiles[1]))
    def _():
        _decode_block(s16_ref[...], anc, strd, wmat, out_ref)

    @pl.when(t >= n_tiles[0] + n_tiles[1])
    def _():
        _decode_block(s32_ref[...], anc, strd, wmat, out_ref)


@jax.jit
def kernel(feat_s8, feat_s16, feat_s32):
    b = feat_s8.shape[0]
    c = 64 + NUM_CLASSES
    f8 = feat_s8.reshape(b, SHAPES[0][0] * SHAPES[0][1], c)
    f16 = feat_s16.reshape(b, SHAPES[1][0] * SHAPES[1][1], c)
    f32_ = feat_s32.reshape(b, SHAPES[2][0] * SHAPES[2][1], c)

    n_tiles = tuple(h * w // ANCHOR_TILE for (h, w) in SHAPES)
    total_tiles = sum(n_tiles)
    n_anchors = ANCHOR_TILE * total_tiles

    bb = 8 if b % 8 == 0 else 1
    grid = (b // bb, total_tiles)

    anchors = jnp.asarray(_ANCHORS)
    strides = jnp.asarray(_STRIDES)
    wmat = jnp.asarray(_WMAT)

    t0, t1, _ = n_tiles
    in_specs = [
        pl.BlockSpec((bb, ANCHOR_TILE, c),
                     lambda i, t: (i, jnp.minimum(t, t0 - 1), 0)),
        pl.BlockSpec((bb, ANCHOR_TILE, c),
                     lambda i, t: (i, jnp.clip(t - t0, 0, t1 - 1), 0)),
        pl.BlockSpec((bb, ANCHOR_TILE, c), lambda i, t: (i, 0, 0)),
        pl.BlockSpec((ANCHOR_TILE, 2), lambda i, t: (t, 0)),
        pl.BlockSpec((ANCHOR_TILE, 1), lambda i, t: (t, 0)),
        pl.BlockSpec((64, 8), lambda i, t: (0, 0)),
    ]
    out_spec = pl.BlockSpec((bb, ANCHOR_TILE, 4 + NUM_CLASSES),
                            lambda i, t: (i, t, 0))

    return pl.pallas_call(
        functools.partial(_body, n_tiles),
        grid=grid,
        in_specs=in_specs,
        out_specs=out_spec,
        out_shape=jax.ShapeDtypeStruct((b, n_anchors, 4 + NUM_CLASSES),
                                       jnp.float32),
    )(f8, f16, f32_, anchors, strides, wmat)


# speedup vs baseline: 1.1135x; 1.1135x over previous
import functools

import jax
import jax.numpy as jnp
import numpy as np
from jax.experimental import pallas as pl

NUM_CLASSES = 80
REG_MAX = 16
STRIDES = (8, 16, 32)
SHAPES = ((80, 80), (40, 40), (20, 20))
C_IN = 64 + NUM_CLASSES
C_OUT = 4 + NUM_CLASSES

ANCHOR_TILE = 400


def _host_constants():
    anchor_rows = []
    for (h, w), s in zip(SHAPES, STRIDES):
        xs = np.arange(w, dtype=np.float32) + 0.5
        ys = np.arange(h, dtype=np.float32) + 0.5
        gx = np.broadcast_to(xs[None, :], (h, w)).reshape(-1)
        gy = np.broadcast_to(ys[:, None], (h, w)).reshape(-1)
        anchor_rows.append(np.stack([gx, gy], axis=1))
    anchors = np.concatenate(anchor_rows, axis=0)

    wmat_t = np.zeros((8, 64), dtype=np.float32)
    for c in range(64):
        side, r = divmod(c, REG_MAX)
        wmat_t[side, c] = float(r)
        wmat_t[4 + side, c] = 1.0

    sa = np.zeros((8, C_OUT), dtype=np.float32)
    for i in range(4):
        sa[i, i] = 1.0

    sc = np.zeros((C_IN, C_OUT), dtype=np.float32)
    for j in range(NUM_CLASSES):
        sc[64 + j, 4 + j] = 1.0
    return anchors, wmat_t, sa, sc


_ANCHORS, _WMAT_T, _SA, _SC = _host_constants()


def _anchors_t(bb):
    tiles = []
    n_tiles = 8400 // ANCHOR_TILE
    for t in range(n_tiles):
        a = _ANCHORS[t * ANCHOR_TILE:(t + 1) * ANCHOR_TILE]
        blk = np.zeros((8, bb * ANCHOR_TILE), dtype=np.float32)
        blk[0] = np.tile(a[:, 0], bb)
        blk[1] = np.tile(a[:, 1], bb)
        tiles.append(blk)
    return np.concatenate(tiles, axis=1)


def _body(bb, t01, s8_ref, s16_ref, s32_ref, anc_ref, wt_ref, sa_ref, sc_ref,
          out_ref):
    t = pl.program_id(1)
    t0, t1 = t01
    stride = jnp.where(t < t0, float(STRIDES[0]),
                       jnp.where(t < t0 + t1, float(STRIDES[1]),
                                 float(STRIDES[2])))
    n = bb * ANCHOR_TILE

    def process(x3):
        x2 = x3.reshape(n, C_IN)
        e = jnp.exp(x2)
        sig = e / (1.0 + e)
        r_t = jax.lax.dot_general(
            wt_ref[...], e[:, :64],
            (((1,), (1,)), ((), ())),
            preferred_element_type=jnp.float32,
        )
        rr = 1.0 / r_t
        dist = r_t * jnp.roll(rr, 4, axis=0)
        summ = dist + jnp.roll(dist, 2, axis=0)
        diff = (jnp.roll(dist, -2, axis=0) - dist) * 0.5
        rows = jax.lax.broadcasted_iota(jnp.int32, (8, n), 0)
        out4 = anc_ref[...] + jnp.where(rows < 2, diff, summ)
        box84 = jax.lax.dot_general(
            out4, sa_ref[...] * stride,
            (((0,), (0,)), ((), ())),
            preferred_element_type=jnp.float32,
        )
        cls84 = jax.lax.dot_general(
            sig, sc_ref[...],
            (((1,), (0,)), ((), ())),
            preferred_element_type=jnp.float32,
        )
        out_ref[...] = (box84 + cls84).reshape(bb, ANCHOR_TILE, C_OUT)

    @pl.when(t < t0)
    def _():
        process(s8_ref[...])

    @pl.when(jnp.logical_and(t >= t0, t < t0 + t1))
    def _():
        process(s16_ref[...])

    @pl.when(t >= t0 + t1)
    def _():
        process(s32_ref[...])


@jax.jit
def kernel(feat_s8, feat_s16, feat_s32):
    b = feat_s8.shape[0]
    f8 = feat_s8.reshape(b, SHAPES[0][0] * SHAPES[0][1], C_IN)
    f16 = feat_s16.reshape(b, SHAPES[1][0] * SHAPES[1][1], C_IN)
    f32_ = feat_s32.reshape(b, SHAPES[2][0] * SHAPES[2][1], C_IN)

    n_tiles = tuple(h * w // ANCHOR_TILE for (h, w) in SHAPES)
    total_tiles = sum(n_tiles)
    n_anchors = ANCHOR_TILE * total_tiles

    bb = 8 if b % 8 == 0 else 1
    grid = (b // bb, total_tiles)

    anc_t = jnp.asarray(_anchors_t(bb))
    wmat_t = jnp.asarray(_WMAT_T)
    sa = jnp.asarray(_SA)
    sc = jnp.asarray(_SC)

    t0, t1, _ = n_tiles
    in_specs = [
        pl.BlockSpec((bb, ANCHOR_TILE, C_IN),
                     lambda i, t: (i, jnp.minimum(t, t0 - 1), 0)),
        pl.BlockSpec((bb, ANCHOR_TILE, C_IN),
                     lambda i, t: (i, jnp.clip(t - t0, 0, t1 - 1), 0)),
        pl.BlockSpec((bb, ANCHOR_TILE, C_IN), lambda i, t: (i, 0, 0)),
        pl.BlockSpec((8, bb * ANCHOR_TILE), lambda i, t: (0, t)),
        pl.BlockSpec((8, 64), lambda i, t: (0, 0)),
        pl.BlockSpec((8, C_OUT), lambda i, t: (0, 0)),
        pl.BlockSpec((C_IN, C_OUT), lambda i, t: (0, 0)),
    ]
    out_spec = pl.BlockSpec((bb, ANCHOR_TILE, C_OUT),
                            lambda i, t: (i, t, 0))

    return pl.pallas_call(
        functools.partial(_body, bb, (t0, t1)),
        grid=grid,
        in_specs=in_specs,
        out_specs=out_spec,
        out_shape=jax.ShapeDtypeStruct((b, n_anchors, C_OUT), jnp.float32),
    )(f8, f16, f32_, anc_t, wmat_t, sa, sc)


# speedup vs baseline: 2.9240x; 2.6259x over previous
import functools

import jax
import jax.numpy as jnp
import numpy as np
from jax.experimental import pallas as pl

NUM_CLASSES = 80
REG_MAX = 16
STRIDES = (8, 16, 32)
SHAPES = ((80, 80), (40, 40), (20, 20))
C_IN = 64 + NUM_CLASSES
C_OUT = 4 + NUM_CLASSES

ANCHOR_TILE = 400


def _host_constants():
    anchor_rows = []
    for (h, w), s in zip(SHAPES, STRIDES):
        xs = np.arange(w, dtype=np.float32) + 0.5
        ys = np.arange(h, dtype=np.float32) + 0.5
        gx = np.broadcast_to(xs[None, :], (h, w)).reshape(-1)
        gy = np.broadcast_to(ys[:, None], (h, w)).reshape(-1)
        anchor_rows.append(np.stack([gx, gy], axis=1))
    anchors = np.concatenate(anchor_rows, axis=0)

    wmat_t = np.zeros((8, 64), dtype=np.float32)
    for c in range(64):
        side, r = divmod(c, REG_MAX)
        wmat_t[side, c] = float(r)
        wmat_t[4 + side, c] = 1.0

    sa = np.zeros((8, C_OUT), dtype=np.float32)
    for i in range(4):
        sa[i, i] = 1.0

    sc = np.zeros((C_IN, C_OUT), dtype=np.float32)
    for j in range(NUM_CLASSES):
        sc[64 + j, 4 + j] = 1.0
    return anchors, wmat_t, sa, sc


_ANCHORS, _WMAT_T, _SA, _SC = _host_constants()


def _anchors_t(bb):
    tiles = []
    n_tiles = 8400 // ANCHOR_TILE
    for t in range(n_tiles):
        a = _ANCHORS[t * ANCHOR_TILE:(t + 1) * ANCHOR_TILE]
        blk = np.zeros((8, bb * ANCHOR_TILE), dtype=np.float32)
        blk[0] = np.tile(a[:, 0], bb)
        blk[1] = np.tile(a[:, 1], bb)
        tiles.append(blk)
    return np.concatenate(tiles, axis=1)


def _body(bb, t01, s8_ref, s16_ref, s32_ref, anc_ref, wt_ref, sa_ref, sc_ref,
          out_ref):
    t = pl.program_id(1)
    t0, t1 = t01
    stride = jnp.where(t < t0, float(STRIDES[0]),
                       jnp.where(t < t0 + t1, float(STRIDES[1]),
                                 float(STRIDES[2])))
    n = bb * ANCHOR_TILE

    def process(x4):
        x2 = x4.reshape(n, C_IN)
        e = jnp.exp(x2)
        sig = e / (1.0 + e)
        r_t = jax.lax.dot_general(
            wt_ref[...], e[:, :64],
            (((1,), (1,)), ((), ())),
            preferred_element_type=jnp.float32,
        )
        rr = 1.0 / r_t
        dist = r_t * jnp.roll(rr, 4, axis=0)
        summ = dist + jnp.roll(dist, 2, axis=0)
        diff = (jnp.roll(dist, -2, axis=0) - dist) * 0.5
        rows = jax.lax.broadcasted_iota(jnp.int32, (8, n), 0)
        out4 = anc_ref[...] + jnp.where(rows < 2, diff, summ)
        box84 = jax.lax.dot_general(
            out4, sa_ref[...] * stride,
            (((0,), (0,)), ((), ())),
            preferred_element_type=jnp.float32,
        )
        cls84 = jax.lax.dot_general(
            sig, sc_ref[...],
            (((1,), (0,)), ((), ())),
            preferred_element_type=jnp.float32,
        )
        out_ref[...] = (box84 + cls84).reshape(bb, ANCHOR_TILE, C_OUT)

    @pl.when(t < t0)
    def _():
        process(s8_ref[...])

    @pl.when(jnp.logical_and(t >= t0, t < t0 + t1))
    def _():
        process(s16_ref[...])

    @pl.when(t >= t0 + t1)
    def _():
        process(s32_ref[...])


@jax.jit
def kernel(feat_s8, feat_s16, feat_s32):
    b = feat_s8.shape[0]

    n_tiles = tuple(h * w // ANCHOR_TILE for (h, w) in SHAPES)
    total_tiles = sum(n_tiles)
    n_anchors = ANCHOR_TILE * total_tiles

    bb = 8 if b % 8 == 0 else 1
    grid = (b // bb, total_tiles)

    anc_t = jnp.asarray(_anchors_t(bb))
    wmat_t = jnp.asarray(_WMAT_T)
    sa = jnp.asarray(_SA)
    sc = jnp.asarray(_SC)

    t0, t1, _ = n_tiles
    r8 = ANCHOR_TILE // SHAPES[0][1]
    r16 = ANCHOR_TILE // SHAPES[1][1]
    r32 = ANCHOR_TILE // SHAPES[2][1]
    in_specs = [
        pl.BlockSpec((bb, r8, SHAPES[0][1], C_IN),
                     lambda i, t: (i, jnp.minimum(t, t0 - 1), 0, 0)),
        pl.BlockSpec((bb, r16, SHAPES[1][1], C_IN),
                     lambda i, t: (i, jnp.clip(t - t0, 0, t1 - 1), 0, 0)),
        pl.BlockSpec((bb, r32, SHAPES[2][1], C_IN),
                     lambda i, t: (i, 0, 0, 0)),
        pl.BlockSpec((8, bb * ANCHOR_TILE), lambda i, t: (0, t)),
        pl.BlockSpec((8, 64), lambda i, t: (0, 0)),
        pl.BlockSpec((8, C_OUT), lambda i, t: (0, 0)),
        pl.BlockSpec((C_IN, C_OUT), lambda i, t: (0, 0)),
    ]
    out_spec = pl.BlockSpec((bb, ANCHOR_TILE, C_OUT),
                            lambda i, t: (i, t, 0))

    return pl.pallas_call(
        functools.partial(_body, bb, (t0, t1)),
        grid=grid,
        in_specs=in_specs,
        out_specs=out_spec,
        out_shape=jax.ShapeDtypeStruct((b, n_anchors, C_OUT), jnp.float32),
    )(feat_s8, feat_s16, feat_s32, anc_t, wmat_t, sa, sc)


# speedup vs baseline: 4.6313x; 1.5839x over previous
import functools

import jax
import jax.numpy as jnp
import numpy as np
from jax.experimental import pallas as pl

NUM_CLASSES = 80
REG_MAX = 16
STRIDES = (8, 16, 32)
SHAPES = ((80, 80), (40, 40), (20, 20))
C_IN = 64 + NUM_CLASSES
C_OUT = 4 + NUM_CLASSES

ANCHOR_TILE = 400


def _host_constants():
    anchor_rows = []
    for (h, w), s in zip(SHAPES, STRIDES):
        xs = np.arange(w, dtype=np.float32) + 0.5
        ys = np.arange(h, dtype=np.float32) + 0.5
        gx = np.broadcast_to(xs[None, :], (h, w)).reshape(-1)
        gy = np.broadcast_to(ys[:, None], (h, w)).reshape(-1)
        anchor_rows.append(np.stack([gx, gy], axis=1))
    anchors = np.concatenate(anchor_rows, axis=0)

    wmat_t = np.zeros((8, 64), dtype=np.float32)
    for c in range(64):
        side, r = divmod(c, REG_MAX)
        wmat_t[side, c] = float(r)
        wmat_t[4 + side, c] = 1.0

    sa = np.zeros((8, C_OUT), dtype=np.float32)
    for i in range(4):
        sa[i, i] = 1.0

    sc = np.zeros((C_IN, C_OUT), dtype=np.float32)
    for j in range(NUM_CLASSES):
        sc[64 + j, 4 + j] = 1.0
    return anchors, wmat_t, sa, sc


_ANCHORS, _WMAT_T, _SA, _SC = _host_constants()


def _anchors_t(bb):
    tiles = []
    n_tiles = 8400 // ANCHOR_TILE
    for t in range(n_tiles):
        a = _ANCHORS[t * ANCHOR_TILE:(t + 1) * ANCHOR_TILE]
        blk = np.zeros((8, bb * ANCHOR_TILE), dtype=np.float32)
        blk[0] = np.tile(a[:, 0], bb)
        blk[1] = np.tile(a[:, 1], bb)
        tiles.append(blk)
    return np.concatenate(tiles, axis=1)


def _body(bb, t01, s8_ref, s16_ref, s32_ref, anc_ref, wt_ref, sa_ref, sc_ref,
          out_ref):
    t = pl.program_id(1)
    t0, t1 = t01
    stride = jnp.where(t < t0, float(STRIDES[0]),
                       jnp.where(t < t0 + t1, float(STRIDES[1]),
                                 float(STRIDES[2])))
    n = bb * ANCHOR_TILE

    def process(x2):
        e = jnp.exp(x2)
        sig = e / (1.0 + e)
        r_t = jax.lax.dot_general(
            wt_ref[...], e[:, :64],
            (((1,), (1,)), ((), ())),
            preferred_element_type=jnp.float32,
        )
        rr = 1.0 / r_t
        dist = r_t * jnp.roll(rr, 4, axis=0)
        summ = dist + jnp.roll(dist, 2, axis=0)
        diff = (jnp.roll(dist, -2, axis=0) - dist) * 0.5
        rows = jax.lax.broadcasted_iota(jnp.int32, (8, n), 0)
        out4 = anc_ref[...] + jnp.where(rows < 2, diff, summ)
        box84 = jax.lax.dot_general(
            out4, sa_ref[...] * stride,
            (((0,), (0,)), ((), ())),
            preferred_element_type=jnp.float32,
        )
        cls84 = jax.lax.dot_general(
            sig, sc_ref[...],
            (((1,), (0,)), ((), ())),
            preferred_element_type=jnp.float32,
        )
        out_ref[...] = (box84 + cls84).reshape(bb, ANCHOR_TILE, C_OUT)

    @pl.when(t < t0)
    def _():
        x = jnp.transpose(s8_ref[...], (0, 1, 3, 2))
        process(x.reshape(n, C_IN))

    @pl.when(jnp.logical_and(t >= t0, t < t0 + t1))
    def _():
        process(s16_ref[...].reshape(n, C_IN))

    @pl.when(t >= t0 + t1)
    def _():
        x = jnp.transpose(s32_ref[...], (2, 0, 1, 3))
        process(x.reshape(n, C_IN))


@jax.jit
def kernel(feat_s8, feat_s16, feat_s32):
    b = feat_s8.shape[0]

    n_tiles = tuple(h * w // ANCHOR_TILE for (h, w) in SHAPES)
    total_tiles = sum(n_tiles)
    n_anchors = ANCHOR_TILE * total_tiles

    bb = 8 if b % 8 == 0 else 1
    grid = (b // bb, total_tiles)

    anc_t = jnp.asarray(_anchors_t(bb))
    wmat_t = jnp.asarray(_WMAT_T)
    sa = jnp.asarray(_SA)
    sc = jnp.asarray(_SC)

    t0, t1, _ = n_tiles
    r8 = ANCHOR_TILE // SHAPES[0][1]
    r16 = ANCHOR_TILE // SHAPES[1][1]
    r32 = ANCHOR_TILE // SHAPES[2][1]

    t8 = jnp.transpose(feat_s8, (0, 1, 3, 2))
    t32 = jnp.transpose(feat_s32, (1, 2, 0, 3))

    in_specs = [
        pl.BlockSpec((bb, r8, C_IN, SHAPES[0][1]),
                     lambda i, t: (i, jnp.minimum(t, t0 - 1), 0, 0)),
        pl.BlockSpec((bb, r16, SHAPES[1][1], C_IN),
                     lambda i, t: (i, jnp.clip(t - t0, 0, t1 - 1), 0, 0)),
        pl.BlockSpec((r32, SHAPES[2][1], bb, C_IN),
                     lambda i, t: (0, 0, i, 0)),
        pl.BlockSpec((8, bb * ANCHOR_TILE), lambda i, t: (0, t)),
        pl.BlockSpec((8, 64), lambda i, t: (0, 0)),
        pl.BlockSpec((8, C_OUT), lambda i, t: (0, 0)),
        pl.BlockSpec((C_IN, C_OUT), lambda i, t: (0, 0)),
    ]
    out_spec = pl.BlockSpec((bb, ANCHOR_TILE, C_OUT),
                            lambda i, t: (i, t, 0))

    return pl.pallas_call(
        functools.partial(_body, bb, (t0, t1)),
        grid=grid,
        in_specs=in_specs,
        out_specs=out_spec,
        out_shape=jax.ShapeDtypeStruct((b, n_anchors, C_OUT), jnp.float32),
    )(t8, feat_s16, t32, anc_t, wmat_t, sa, sc)
